# baseline (speedup 1.0000x reference)
import functools

import jax
import jax.numpy as jnp
from jax import lax
from jax.experimental import pallas as pl
from jax.experimental.pallas import tpu as pltpu
from jax.experimental.pallas import tpu_sc as plsc

N, D, P, HD = 10000, 128, 12, 128
E = 320000

NC, NS = 2, 16
K = 100
NB = E // (NS * K)
NBD = E // (NC * NS * K)
PPC = P // NC
SPB = 640
CH = 40
NBI = 40
NCI = NB // NBI

NBLK = 1000
GRID = N // NBLK


def _deg_body(dst_hbm, deg_hbm, idx_v, ones_v, zrow_v, deg_sp):
    c = lax.axis_index("c")
    s = lax.axis_index("s")

    def fill_ones(i, carry):
        for t in range(D // 16):
            ones_v[i, pl.ds(t * 16, 16)] = jnp.ones((16,), jnp.float32)
        return carry
    lax.fori_loop(0, K, fill_ones, 0)

    def fill_zero(i, carry):
        for t in range(D // 16):
            zrow_v[i, pl.ds(t * 16, 16)] = jnp.zeros((16,), jnp.float32)
        return carry
    lax.fori_loop(0, CH, fill_zero, 0)

    base = s * SPB
    nq = jnp.where(s == NS - 1, (N - (NS - 1) * SPB) // CH, SPB // CH)

    def zchunk(q, carry):
        pltpu.sync_copy(zrow_v, deg_sp.at[pl.ds(base + q * CH, CH)])
        return carry
    lax.fori_loop(0, nq, zchunk, 0)
    plsc.subcore_barrier()

    pltpu.sync_copy(dst_hbm.at[c].at[s], idx_v)

    def scat(j, carry):
        pltpu.sync_copy(ones_v, deg_sp.at[idx_v.at[j]], add=True)
        return carry
    lax.fori_loop(0, NBD, scat, 0)
    plsc.subcore_barrier()

    def dchunk(q, carry):
        pltpu.sync_copy(deg_sp.at[pl.ds(base + q * CH, CH)],
                        deg_hbm.at[c].at[pl.ds(base + q * CH, CH)])
        return carry
    lax.fori_loop(0, nq, dchunk, 0)


def _degree_count(dst_batched):
    k = functools.partial(
        pl.kernel,
        mesh=plsc.VectorSubcoreMesh(core_axis_name="c", subcore_axis_name="s"),
        out_type=jax.ShapeDtypeStruct((NC, N, D), jnp.float32),
        scratch_types=[
            pltpu.VMEM((NBD, K), jnp.int32),
            pltpu.VMEM((K, D), jnp.float32),
            pltpu.VMEM((CH, D), jnp.float32),
            pltpu.VMEM_SHARED((N, D), jnp.float32),
        ],
    )(_deg_body)
    return k(dst_batched)


def _prop_body(xp_hbm, src_hbm, dst_hbm, s_out_hbm,
               src_v, dst_v, row0_v, row1_v, zbuf_v, obuf_v, sem0, sem1, y_sp):
    c = lax.axis_index("c")
    s = lax.axis_index("s")

    def fill_zero(i, carry):
        for t in range(D // 16):
            zbuf_v[i, pl.ds(t * 16, 16)] = jnp.zeros((16,), jnp.float32)
        return carry
    lax.fori_loop(0, CH, fill_zero, 0)

    base = s * SPB
    nq = jnp.where(s == NS - 1, (N - (NS - 1) * SPB) // CH, SPB // CH)

    for p_local in range(PPC):
        p = c * PPC + p_local

        def zchunk(q, carry):
            pltpu.sync_copy(zbuf_v, y_sp.at[pl.ds(base + q * CH, CH)])
            return carry
        lax.fori_loop(0, nq, zchunk, 0)
        plsc.subcore_barrier()

        def idx_chunk(ci, carry):
            pltpu.sync_copy(src_hbm.at[s].at[pl.ds(ci * NBI, NBI)], src_v)
            pltpu.sync_copy(dst_hbm.at[s].at[pl.ds(ci * NBI, NBI)], dst_v)

            pltpu.async_copy(xp_hbm.at[p].at[src_v.at[0]], row0_v, sem0)

            def pair(t, carry2):
                pltpu.async_copy(xp_hbm.at[p].at[src_v.at[2 * t + 1]],
                                 row1_v, sem1)
                pltpu.make_async_copy(xp_hbm.at[p].at[src_v.at[2 * t]],
                                      row0_v, sem0).wait()
                pltpu.sync_copy(row0_v, y_sp.at[dst_v.at[2 * t]], add=True)

                @pl.when(t < NBI // 2 - 1)
                def _():
                    pltpu.async_copy(xp_hbm.at[p].at[src_v.at[2 * t + 2]],
                                     row0_v, sem0)
                pltpu.make_async_copy(xp_hbm.at[p].at[src_v.at[2 * t + 1]],
                                      row1_v, sem1).wait()
                pltpu.sync_copy(row1_v, y_sp.at[dst_v.at[2 * t + 1]], add=True)
                return carry2
            lax.fori_loop(0, NBI // 2, pair, 0)
            return carry
        lax.fori_loop(0, NCI, idx_chunk, 0)
        plsc.subcore_barrier()

        def drain(q, carry):
            pltpu.sync_copy(y_sp.at[pl.ds(base + q * CH, CH)], obuf_v)
            pltpu.sync_copy(obuf_v,
                            s_out_hbm.at[p].at[pl.ds(base + q * CH, CH)])
            return carry
        lax.fori_loop(0, nq, drain, 0)


def _propagate(xp, src_batched, dst_batched):
    k = functools.partial(
        pl.kernel,
        mesh=plsc.VectorSubcoreMesh(core_axis_name="c", subcore_axis_name="s"),
        out_type=jax.ShapeDtypeStruct((P, N, D), jnp.float32),
        scratch_types=[
            pltpu.VMEM((NBI, K), jnp.int32),
            pltpu.VMEM((NBI, K), jnp.int32),
            pltpu.VMEM((K, D), jnp.float32),
            pltpu.VMEM((K, D), jnp.float32),
            pltpu.VMEM((CH, D), jnp.float32),
            pltpu.VMEM((CH, D), jnp.float32),
            pltpu.SemaphoreType.DMA,
            pltpu.SemaphoreType.DMA,
            pltpu.VMEM_SHARED((N, D), jnp.float32),
        ],
    )(_prop_body)
    return k(xp, src_batched, dst_batched)


def _scale_body(xT_ref, degA_ref, degB_ref, xp_ref, dinv_ref):
    deg = degA_ref[...] + degB_ref[...] + 1.0
    di = lax.rsqrt(deg)
    dinv_ref[...] = di
    xp_ref[...] = xT_ref[...] * di[None, :, :]


def _scale(xT, degA, degB):
    return pl.pallas_call(
        _scale_body,
        grid=(GRID,),
        in_specs=[
            pl.BlockSpec((P, NBLK, D), lambda i: (0, i, 0)),
            pl.BlockSpec((NBLK, 1), lambda i: (i, 0)),
            pl.BlockSpec((NBLK, 1), lambda i: (i, 0)),
        ],
        out_specs=[
            pl.BlockSpec((P, NBLK, D), lambda i: (0, i, 0)),
            pl.BlockSpec((NBLK, 1), lambda i: (i, 0)),
        ],
        out_shape=[
            jax.ShapeDtypeStruct((P, N, D), jnp.float32),
            jax.ShapeDtypeStruct((N, 1), jnp.float32),
        ],
    )(xT, degA, degB)


def _gates_body(S_ref, xp_ref, dinv_ref, Wcz_ref, Wlz_ref, bcz_ref, blz_ref,
                Wch_ref, Wlh_ref, bch_ref, blh_ref,
                att_ref, h_ref, stats_ref):
    i = pl.program_id(0)
    di = dinv_ref[...]
    probs = jax.nn.softmax(att_ref[0, :])
    Wlz = Wlz_ref[0:HD, :]
    Wlh = Wlh_ref[0:HD, :]
    Wz = jnp.dot(Wcz_ref[...], Wlz, preferred_element_type=jnp.float32)
    bz = jnp.dot(bcz_ref[...], Wlz, preferred_element_type=jnp.float32) + blz_ref[...]
    Wh = jnp.dot(Wch_ref[...], Wlh, preferred_element_type=jnp.float32)
    bh = jnp.dot(bch_ref[...], Wlh, preferred_element_type=jnp.float32) + blh_ref[...]
    Hacc = jnp.zeros((NBLK, HD), jnp.float32)
    for p in range(P):
        yp = (S_ref[p] + xp_ref[p]) * di
        Z = jax.nn.sigmoid(jnp.dot(yp, Wz, preferred_element_type=jnp.float32) + bz)
        T = jnp.tanh(jnp.dot(yp, Wh, preferred_element_type=jnp.float32) + bh)
        Hacc = Hacc + probs[p] * (1.0 - Z) * T
    h = jnp.maximum(Hacc, 0.0)
    h_ref[...] = h

    @pl.when(i == 0)
    def _():
        stats_ref[...] = jnp.zeros((8, HD), jnp.float32)

    stats_ref[0, :] += jnp.sum(h, axis=0)
    stats_ref[1, :] += jnp.sum(h * h, axis=0)


def _gates(S, xp, dinv, Wcz, Wlz, bcz, blz, Wch, Wlh, bch, blh, att_pad):
    return pl.pallas_call(
        _gates_body,
        grid=(GRID,),
        in_specs=[
            pl.BlockSpec((P, NBLK, D), lambda i: (0, i, 0)),
            pl.BlockSpec((P, NBLK, D), lambda i: (0, i, 0)),
            pl.BlockSpec((NBLK, 1), lambda i: (i, 0)),
            pl.BlockSpec((HD, HD), lambda i: (0, 0)),
            pl.BlockSpec((2 * HD, HD), lambda i: (0, 0)),
            pl.BlockSpec((1, HD), lambda i: (0, 0)),
            pl.BlockSpec((1, HD), lambda i: (0, 0)),
            pl.BlockSpec((HD, HD), lambda i: (0, 0)),
            pl.BlockSpec((2 * HD, HD), lambda i: (0, 0)),
            pl.BlockSpec((1, HD), lambda i: (0, 0)),
            pl.BlockSpec((1, HD), lambda i: (0, 0)),
            pl.BlockSpec((1, 128), lambda i: (0, 0)),
        ],
        out_specs=[
            pl.BlockSpec((NBLK, HD), lambda i: (i, 0)),
            pl.BlockSpec((8, HD), lambda i: (0, 0)),
        ],
        out_shape=[
            jax.ShapeDtypeStruct((N, HD), jnp.float32),
            jax.ShapeDtypeStruct((8, HD), jnp.float32),
        ],
    )(S, xp, dinv, Wcz, Wlz, bcz, blz, Wch, Wlh, bch, blh, att_pad)


def _final_body(h_ref, stats_ref, gamma_ref, beta_ref, Wout_ref, bout_ref,
                out_ref):
    mean = stats_ref[0, :] * (1.0 / N)
    var = stats_ref[1, :] * (1.0 / N) - mean * mean
    scale = gamma_ref[0, :] * lax.rsqrt(var + 1e-5)
    h = (h_ref[...] - mean[None, :]) * scale[None, :] + beta_ref[0, :][None, :]
    out_ref[...] = jnp.dot(h, Wout_ref[...], preferred_element_type=jnp.float32) + bout_ref[...]


def _final(h, stats, gamma, beta, Wout, bout):
    return pl.pallas_call(
        _final_body,
        grid=(GRID,),
        in_specs=[
            pl.BlockSpec((NBLK, HD), lambda i: (i, 0)),
            pl.BlockSpec((8, HD), lambda i: (0, 0)),
            pl.BlockSpec((1, HD), lambda i: (0, 0)),
            pl.BlockSpec((1, HD), lambda i: (0, 0)),
            pl.BlockSpec((HD, P * D), lambda i: (0, 0)),
            pl.BlockSpec((1, P * D), lambda i: (0, 0)),
        ],
        out_specs=pl.BlockSpec((NBLK, P * D), lambda i: (i, 0)),
        out_shape=jax.ShapeDtypeStruct((N, P * D), jnp.float32),
    )(h, stats, gamma, beta, Wout, bout)


def kernel(x, edge_index, W_conv_z, b_conv_z, W_lin_z, b_lin_z,
           W_conv_r, b_conv_r, W_lin_r, b_lin_r,
           W_conv_h, b_conv_h, W_lin_h, b_lin_h,
           attention, bn_gamma, bn_beta, W_out, b_out):
    src = edge_index[0]
    dst = edge_index[1]
    src_batched = src.reshape(NS, NB, K)
    dst_batched = dst.reshape(NS, NB, K)
    dst_deg = dst.reshape(NC, NS, NBD, K)

    xT = jnp.transpose(x, (2, 0, 1))

    deg_raw = _degree_count(dst_deg)
    degA = deg_raw[0, :, 0].reshape(N, 1)
    degB = deg_raw[1, :, 0].reshape(N, 1)

    xp, dinv = _scale(xT, degA, degB)

    S = _propagate(xp, src_batched, dst_batched)

    att_pad = jnp.concatenate(
        [attention.reshape(1, P),
         jnp.full((1, 128 - P), -1e30, jnp.float32)], axis=1)

    h, stats = _gates(S, xp, dinv,
                      W_conv_z, W_lin_z,
                      b_conv_z.reshape(1, HD), b_lin_z.reshape(1, HD),
                      W_conv_h, W_lin_h,
                      b_conv_h.reshape(1, HD), b_lin_h.reshape(1, HD),
                      att_pad)

    out = _final(h, stats, bn_gamma.reshape(1, HD), bn_beta.reshape(1, HD),
                 W_out, b_out.reshape(1, P * D))
    return out.reshape(N, D, P)


# speedup vs baseline: 1.0048x; 1.0048x over previous
import functools

import jax
import jax.numpy as jnp
from jax import lax
from jax.experimental import pallas as pl
from jax.experimental.pallas import tpu as pltpu
from jax.experimental.pallas import tpu_sc as plsc

N, D, P, HD = 10000, 128, 12, 128
E = 320000

NC, NS = 2, 16
K = 100
NB = E // (NS * K)
NBD = E // (NC * NS * K)
PPC = P // NC
SPB = 640
CH = 40
NBI = 40
NCI = NB // NBI

NBLK = 1000
GRID = N // NBLK


def _deg_body(dst_hbm, deg_hbm, idx_v, ones_v, zrow_v, deg_sp):
    c = lax.axis_index("c")
    s = lax.axis_index("s")

    def fill_ones(i, carry):
        for t in range(D // 16):
            ones_v[i, pl.ds(t * 16, 16)] = jnp.ones((16,), jnp.float32)
        return carry
    lax.fori_loop(0, K, fill_ones, 0)

    def fill_zero(i, carry):
        for t in range(D // 16):
            zrow_v[i, pl.ds(t * 16, 16)] = jnp.zeros((16,), jnp.float32)
        return carry
    lax.fori_loop(0, CH, fill_zero, 0)

    base = s * SPB
    nq = jnp.where(s == NS - 1, (N - (NS - 1) * SPB) // CH, SPB // CH)

    def zchunk(q, carry):
        pltpu.sync_copy(zrow_v, deg_sp.at[pl.ds(base + q * CH, CH)])
        return carry
    lax.fori_loop(0, nq, zchunk, 0)
    plsc.subcore_barrier()

    pltpu.sync_copy(dst_hbm.at[c].at[s], idx_v)

    def scat(j, carry):
        pltpu.sync_copy(ones_v, deg_sp.at[idx_v.at[j]], add=True)
        return carry
    lax.fori_loop(0, NBD, scat, 0)
    plsc.subcore_barrier()

    def dchunk(q, carry):
        pltpu.sync_copy(deg_sp.at[pl.ds(base + q * CH, CH)],
                        deg_hbm.at[c].at[pl.ds(base + q * CH, CH)])
        return carry
    lax.fori_loop(0, nq, dchunk, 0)


def _degree_count(dst_batched):
    k = functools.partial(
        pl.kernel,
        mesh=plsc.VectorSubcoreMesh(core_axis_name="c", subcore_axis_name="s"),
        out_type=jax.ShapeDtypeStruct((NC, N, D), jnp.float32),
        scratch_types=[
            pltpu.VMEM((NBD, K), jnp.int32),
            pltpu.VMEM((K, D), jnp.float32),
            pltpu.VMEM((CH, D), jnp.float32),
            pltpu.VMEM_SHARED((N, D), jnp.float32),
        ],
    )(_deg_body)
    return k(dst_batched)


def _prop_body(xp_hbm, src_hbm, dst_hbm, s_out_hbm,
               src_v, dst_v, row0_v, row1_v, zbuf_v, obuf0_v, obuf1_v,
               sem0, sem1, dsem0, dsem1, y_sp):
    c = lax.axis_index("c")
    s = lax.axis_index("s")

    def fill_zero(i, carry):
        for t in range(D // 16):
            zbuf_v[i, pl.ds(t * 16, 16)] = jnp.zeros((16,), jnp.float32)
        return carry
    lax.fori_loop(0, 24, fill_zero, 0)

    base = s * SPB
    nq = jnp.where(s == NS - 1, (N - (NS - 1) * SPB) // CH, SPB // CH)

    def zero_chunk(q):
        pltpu.sync_copy(zbuf_v, y_sp.at[pl.ds(base + q * CH, 24)])
        pltpu.sync_copy(zbuf_v.at[pl.ds(0, 16)],
                        y_sp.at[pl.ds(base + q * CH + 24, 16)])

    for p_local in range(PPC):
        p = c * PPC + p_local

        if p_local == 0:
            def zchunk(q, carry):
                zero_chunk(q)
                return carry
            lax.fori_loop(0, nq, zchunk, 0)
            plsc.subcore_barrier()

        def idx_chunk(ci, carry):
            pltpu.sync_copy(src_hbm.at[s].at[pl.ds(ci * NBI, NBI)], src_v)
            pltpu.sync_copy(dst_hbm.at[s].at[pl.ds(ci * NBI, NBI)], dst_v)

            pltpu.async_copy(xp_hbm.at[p].at[src_v.at[0]], row0_v, sem0)

            def pair(t, carry2):
                pltpu.async_copy(xp_hbm.at[p].at[src_v.at[2 * t + 1]],
                                 row1_v, sem1)
                pltpu.make_async_copy(xp_hbm.at[p].at[src_v.at[2 * t]],
                                      row0_v, sem0).wait()
                pltpu.sync_copy(row0_v, y_sp.at[dst_v.at[2 * t]], add=True)

                @pl.when(t < NBI // 2 - 1)
                def _():
                    pltpu.async_copy(xp_hbm.at[p].at[src_v.at[2 * t + 2]],
                                     row0_v, sem0)
                pltpu.make_async_copy(xp_hbm.at[p].at[src_v.at[2 * t + 1]],
                                      row1_v, sem1).wait()
                pltpu.sync_copy(row1_v, y_sp.at[dst_v.at[2 * t + 1]], add=True)
                return carry2
            lax.fori_loop(0, NBI // 2, pair, 0)
            return carry
        lax.fori_loop(0, NCI, idx_chunk, 0)
        plsc.subcore_barrier()

        def dz(q, obuf, dsem, first):
            @pl.when(jnp.logical_not(first))
            def _():
                pltpu.make_async_copy(
                    obuf, s_out_hbm.at[p].at[pl.ds(base + q * CH, CH)],
                    dsem).wait()
            pltpu.sync_copy(y_sp.at[pl.ds(base + q * CH, CH)], obuf)
            pltpu.async_copy(obuf,
                             s_out_hbm.at[p].at[pl.ds(base + q * CH, CH)],
                             dsem)
            zero_chunk(q)

        def dpair(qq, carry):
            dz(2 * qq, obuf0_v, dsem0, qq == 0)
            dz(2 * qq + 1, obuf1_v, dsem1, qq == 0)
            return carry
        lax.fori_loop(0, nq // 2, dpair, 0)
        pltpu.make_async_copy(obuf0_v, s_out_hbm.at[p].at[pl.ds(base, CH)],
                              dsem0).wait()
        pltpu.make_async_copy(obuf1_v, s_out_hbm.at[p].at[pl.ds(base, CH)],
                              dsem1).wait()

        if p_local < PPC - 1:
            plsc.subcore_barrier()


def _propagate(xp, src_batched, dst_batched):
    k = functools.partial(
        pl.kernel,
        mesh=plsc.VectorSubcoreMesh(core_axis_name="c", subcore_axis_name="s"),
        out_type=jax.ShapeDtypeStruct((P, N, D), jnp.float32),
        scratch_types=[
            pltpu.VMEM((NBI, K), jnp.int32),
            pltpu.VMEM((NBI, K), jnp.int32),
            pltpu.VMEM((K, D), jnp.float32),
            pltpu.VMEM((K, D), jnp.float32),
            pltpu.VMEM((24, D), jnp.float32),
            pltpu.VMEM((CH, D), jnp.float32),
            pltpu.VMEM((CH, D), jnp.float32),
            pltpu.SemaphoreType.DMA,
            pltpu.SemaphoreType.DMA,
            pltpu.SemaphoreType.DMA,
            pltpu.SemaphoreType.DMA,
            pltpu.VMEM_SHARED((N, D), jnp.float32),
        ],
    )(_prop_body)
    return k(xp, src_batched, dst_batched)


def _scale_body(xT_ref, degA_ref, degB_ref, xp_ref, dinv_ref):
    deg = degA_ref[...] + degB_ref[...] + 1.0
    di = lax.rsqrt(deg)
    dinv_ref[...] = di
    xp_ref[...] = xT_ref[...] * di[None, :, :]


def _scale(xT, degA, degB):
    return pl.pallas_call(
        _scale_body,
        grid=(GRID,),
        in_specs=[
            pl.BlockSpec((P, NBLK, D), lambda i: (0, i, 0)),
            pl.BlockSpec((NBLK, 1), lambda i: (i, 0)),
            pl.BlockSpec((NBLK, 1), lambda i: (i, 0)),
        ],
        out_specs=[
            pl.BlockSpec((P, NBLK, D), lambda i: (0, i, 0)),
            pl.BlockSpec((NBLK, 1), lambda i: (i, 0)),
        ],
        out_shape=[
            jax.ShapeDtypeStruct((P, N, D), jnp.float32),
            jax.ShapeDtypeStruct((N, 1), jnp.float32),
        ],
    )(xT, degA, degB)


def _gates_body(S_ref, xp_ref, dinv_ref, Wcz_ref, Wlz_ref, bcz_ref, blz_ref,
                Wch_ref, Wlh_ref, bch_ref, blh_ref,
                att_ref, h_ref, stats_ref):
    i = pl.program_id(0)
    di = dinv_ref[...]
    probs = jax.nn.softmax(att_ref[0, :])
    Wlz = Wlz_ref[0:HD, :]
    Wlh = Wlh_ref[0:HD, :]
    Wz = jnp.dot(Wcz_ref[...], Wlz, preferred_element_type=jnp.float32)
    bz = jnp.dot(bcz_ref[...], Wlz, preferred_element_type=jnp.float32) + blz_ref[...]
    Wh = jnp.dot(Wch_ref[...], Wlh, preferred_element_type=jnp.float32)
    bh = jnp.dot(bch_ref[...], Wlh, preferred_element_type=jnp.float32) + blh_ref[...]
    Hacc = jnp.zeros((NBLK, HD), jnp.float32)
    for p in range(P):
        yp = (S_ref[p] + xp_ref[p]) * di
        Z = jax.nn.sigmoid(jnp.dot(yp, Wz, preferred_element_type=jnp.float32) + bz)
        T = jnp.tanh(jnp.dot(yp, Wh, preferred_element_type=jnp.float32) + bh)
        Hacc = Hacc + probs[p] * (1.0 - Z) * T
    h = jnp.maximum(Hacc, 0.0)
    h_ref[...] = h

    @pl.when(i == 0)
    def _():
        stats_ref[...] = jnp.zeros((8, HD), jnp.float32)

    stats_ref[0, :] += jnp.sum(h, axis=0)
    stats_ref[1, :] += jnp.sum(h * h, axis=0)


def _gates(S, xp, dinv, Wcz, Wlz, bcz, blz, Wch, Wlh, bch, blh, att_pad):
    return pl.pallas_call(
        _gates_body,
        grid=(GRID,),
        in_specs=[
            pl.BlockSpec((P, NBLK, D), lambda i: (0, i, 0)),
            pl.BlockSpec((P, NBLK, D), lambda i: (0, i, 0)),
            pl.BlockSpec((NBLK, 1), lambda i: (i, 0)),
            pl.BlockSpec((HD, HD), lambda i: (0, 0)),
            pl.BlockSpec((2 * HD, HD), lambda i: (0, 0)),
            pl.BlockSpec((1, HD), lambda i: (0, 0)),
            pl.BlockSpec((1, HD), lambda i: (0, 0)),
            pl.BlockSpec((HD, HD), lambda i: (0, 0)),
            pl.BlockSpec((2 * HD, HD), lambda i: (0, 0)),
            pl.BlockSpec((1, HD), lambda i: (0, 0)),
            pl.BlockSpec((1, HD), lambda i: (0, 0)),
            pl.BlockSpec((1, 128), lambda i: (0, 0)),
        ],
        out_specs=[
            pl.BlockSpec((NBLK, HD), lambda i: (i, 0)),
            pl.BlockSpec((8, HD), lambda i: (0, 0)),
        ],
        out_shape=[
            jax.ShapeDtypeStruct((N, HD), jnp.float32),
            jax.ShapeDtypeStruct((8, HD), jnp.float32),
        ],
    )(S, xp, dinv, Wcz, Wlz, bcz, blz, Wch, Wlh, bch, blh, att_pad)


def _final_body(h_ref, stats_ref, gamma_ref, beta_ref, Wout_ref, bout_ref,
                out_ref):
    mean = stats_ref[0, :] * (1.0 / N)
    var = stats_ref[1, :] * (1.0 / N) - mean * mean
    scale = gamma_ref[0, :] * lax.rsqrt(var + 1e-5)
    h = (h_ref[...] - mean[None, :]) * scale[None, :] + beta_ref[0, :][None, :]
    out_ref[...] = jnp.dot(h, Wout_ref[...], preferred_element_type=jnp.float32) + bout_ref[...]


def _final(h, stats, gamma, beta, Wout, bout):
    return pl.pallas_call(
        _final_body,
        grid=(GRID,),
        in_specs=[
            pl.BlockSpec((NBLK, HD), lambda i: (i, 0)),
            pl.BlockSpec((8, HD), lambda i: (0, 0)),
            pl.BlockSpec((1, HD), lambda i: (0, 0)),
            pl.BlockSpec((1, HD), lambda i: (0, 0)),
            pl.BlockSpec((HD, P * D), lambda i: (0, 0)),
            pl.BlockSpec((1, P * D), lambda i: (0, 0)),
        ],
        out_specs=pl.BlockSpec((NBLK, P * D), lambda i: (i, 0)),
        out_shape=jax.ShapeDtypeStruct((N, P * D), jnp.float32),
    )(h, stats, gamma, beta, Wout, bout)


def kernel(x, edge_index, W_conv_z, b_conv_z, W_lin_z, b_lin_z,
           W_conv_r, b_conv_r, W_lin_r, b_lin_r,
           W_conv_h, b_conv_h, W_lin_h, b_lin_h,
           attention, bn_gamma, bn_beta, W_out, b_out):
    src = edge_index[0]
    dst = edge_index[1]
    src_batched = src.reshape(NS, NB, K)
    dst_batched = dst.reshape(NS, NB, K)
    dst_deg = dst.reshape(NC, NS, NBD, K)

    xT = jnp.transpose(x, (2, 0, 1))

    deg_raw = _degree_count(dst_deg)
    degA = deg_raw[0, :, 0].reshape(N, 1)
    degB = deg_raw[1, :, 0].reshape(N, 1)

    xp, dinv = _scale(xT, degA, degB)

    S = _propagate(xp, src_batched, dst_batched)

    att_pad = jnp.concatenate(
        [attention.reshape(1, P),
         jnp.full((1, 128 - P), -1e30, jnp.float32)], axis=1)

    h, stats = _gates(S, xp, dinv,
                      W_conv_z, W_lin_z,
                      b_conv_z.reshape(1, HD), b_lin_z.reshape(1, HD),
                      W_conv_h, W_lin_h,
                      b_conv_h.reshape(1, HD), b_lin_h.reshape(1, HD),
                      att_pad)

    out = _final(h, stats, bn_gamma.reshape(1, HD), bn_beta.reshape(1, HD),
                 W_out, b_out.reshape(1, P * D))
    return out.reshape(N, D, P)
